# baseline (speedup 1.0000x reference)
import functools

import jax
import jax.numpy as jnp
from jax import lax
from jax.experimental import pallas as pl
from jax.experimental.pallas import tpu as pltpu
from jax.experimental.pallas import tpu_sc as plsc

L_BITS = 20
BATCH = 16384
NUM_CORES = 2
NUM_SUBCORES = 16
NUM_WORKERS = NUM_CORES * NUM_SUBCORES
B_W = BATCH // NUM_WORKERS
CHUNK = 128
N_CHUNKS = B_W // CHUNK
TC_BLOCK = 2048


def _index_body(x_ref, idx_ref):
    x = x_ref[...]
    j = lax.broadcasted_iota(jnp.int32, (1, L_BITS), 1)
    w = jnp.left_shift(jnp.int32(1), (L_BITS - 1) - j)
    contrib = jnp.where(x > 0.0, w, jnp.zeros_like(w))
    idx_ref[...] = jnp.sum(contrib, axis=1)


def _gather_body(idx_hbm, aux_hbm, out_hbm, idx_v, out_v, sem):
    cid = lax.axis_index("c")
    sid = lax.axis_index("s")
    wid = sid * NUM_CORES + cid
    base = wid * B_W

    pltpu.sync_copy(idx_hbm.at[pl.ds(base, B_W)], idx_v)

    copies = []
    for c in range(N_CHUNKS):
        copies.append(
            pltpu.async_copy(
                aux_hbm.at[idx_v.at[pl.ds(c * CHUNK, CHUNK)]],
                out_v.at[pl.ds(c * CHUNK, CHUNK)],
                sem,
            )
        )
    for cp in copies:
        cp.wait()

    pltpu.sync_copy(out_v, out_hbm.at[pl.ds(base, B_W)])


@jax.jit
def kernel(x, aux):
    indices = pl.pallas_call(
        _index_body,
        grid=(BATCH // TC_BLOCK,),
        in_specs=[pl.BlockSpec((TC_BLOCK, L_BITS), lambda i: (i, 0))],
        out_specs=pl.BlockSpec((TC_BLOCK,), lambda i: (i,)),
        out_shape=jax.ShapeDtypeStruct((BATCH,), jnp.int32),
    )(x)

    mesh = plsc.VectorSubcoreMesh(core_axis_name="c", subcore_axis_name="s")
    run = pl.kernel(
        _gather_body,
        out_type=jax.ShapeDtypeStruct((BATCH,), jnp.float32),
        mesh=mesh,
        compiler_params=pltpu.CompilerParams(needs_layout_passes=False),
        scratch_types=[
            pltpu.VMEM((B_W,), jnp.int32),
            pltpu.VMEM((B_W,), jnp.float32),
            pltpu.SemaphoreType.DMA,
        ],
    )
    return run(indices, aux)


# speedup vs baseline: 1.6983x; 1.6983x over previous
import functools

import jax
import jax.numpy as jnp
from jax import lax
from jax.experimental import pallas as pl
from jax.experimental.pallas import tpu as pltpu
from jax.experimental.pallas import tpu_sc as plsc

L_BITS = 20
BATCH = 16384
NUM_CORES = 2
NUM_SUBCORES = 16
NUM_WORKERS = NUM_CORES * NUM_SUBCORES
B_W = BATCH // NUM_WORKERS
CHUNK = 128
N_CHUNKS = B_W // CHUNK
TC_BLOCK = 4096


def _index_body(xt_ref, idx_ref):
    xt = xt_ref[...]
    j = lax.broadcasted_iota(jnp.int32, (L_BITS, 1), 0)
    w = jnp.left_shift(jnp.int32(1), (L_BITS - 1) - j)
    contrib = jnp.where(xt > 0.0, w, jnp.zeros_like(w))
    idx_ref[...] = jnp.sum(contrib, axis=0)


def _gather_body(idx_hbm, aux_hbm, out_hbm, idx_v, out_v, sem):
    cid = lax.axis_index("c")
    sid = lax.axis_index("s")
    wid = sid * NUM_CORES + cid
    base = wid * B_W

    pltpu.sync_copy(idx_hbm.at[pl.ds(base, B_W)], idx_v)

    copies = []
    for c in range(N_CHUNKS):
        copies.append(
            pltpu.async_copy(
                aux_hbm.at[idx_v.at[pl.ds(c * CHUNK, CHUNK)]],
                out_v.at[pl.ds(c * CHUNK, CHUNK)],
                sem,
            )
        )
    for cp in copies:
        cp.wait()

    pltpu.sync_copy(out_v, out_hbm.at[pl.ds(base, B_W)])


@jax.jit
def kernel(x, aux):
    indices = pl.pallas_call(
        _index_body,
        grid=(BATCH // TC_BLOCK,),
        in_specs=[pl.BlockSpec((L_BITS, TC_BLOCK), lambda i: (0, i))],
        out_specs=pl.BlockSpec((TC_BLOCK,), lambda i: (i,)),
        out_shape=jax.ShapeDtypeStruct((BATCH,), jnp.int32),
    )(x.T)

    mesh = plsc.VectorSubcoreMesh(core_axis_name="c", subcore_axis_name="s")
    run = pl.kernel(
        _gather_body,
        out_type=jax.ShapeDtypeStruct((BATCH,), jnp.float32),
        mesh=mesh,
        compiler_params=pltpu.CompilerParams(needs_layout_passes=False),
        scratch_types=[
            pltpu.VMEM((B_W,), jnp.int32),
            pltpu.VMEM((B_W,), jnp.float32),
            pltpu.SemaphoreType.DMA,
        ],
    )
    return run(indices, aux)


# speedup vs baseline: 1.7623x; 1.0377x over previous
import functools

import jax
import jax.numpy as jnp
from jax import lax
from jax.experimental import pallas as pl
from jax.experimental.pallas import tpu as pltpu
from jax.experimental.pallas import tpu_sc as plsc

L_BITS = 20
BATCH = 16384
NUM_CORES = 2
NUM_SUBCORES = 16
NUM_WORKERS = NUM_CORES * NUM_SUBCORES
B_W = BATCH // NUM_WORKERS
CHUNK = 128
N_CHUNKS = B_W // CHUNK
TC_BLOCK = 16384


def _index_body(xt_ref, idx_ref):
    xt = xt_ref[...]
    j = lax.broadcasted_iota(jnp.int32, (L_BITS, 1), 0)
    w = jnp.left_shift(jnp.int32(1), (L_BITS - 1) - j)
    contrib = jnp.where(xt > 0.0, w, jnp.zeros_like(w))
    idx_ref[...] = jnp.sum(contrib, axis=0)


def _gather_body(idx_hbm, aux_hbm, out_hbm, idx_v, out_v, sem):
    cid = lax.axis_index("c")
    sid = lax.axis_index("s")
    wid = sid * NUM_CORES + cid
    base = wid * B_W

    pltpu.sync_copy(idx_hbm.at[pl.ds(base, B_W)], idx_v)

    copies = []
    for c in range(N_CHUNKS):
        copies.append(
            pltpu.async_copy(
                aux_hbm.at[idx_v.at[pl.ds(c * CHUNK, CHUNK)]],
                out_v.at[pl.ds(c * CHUNK, CHUNK)],
                sem,
            )
        )
    for cp in copies:
        cp.wait()

    pltpu.sync_copy(out_v, out_hbm.at[pl.ds(base, B_W)])


@jax.jit
def kernel(x, aux):
    indices = pl.pallas_call(
        _index_body,
        grid=(BATCH // TC_BLOCK,),
        in_specs=[pl.BlockSpec((L_BITS, TC_BLOCK), lambda i: (0, i))],
        out_specs=pl.BlockSpec((TC_BLOCK,), lambda i: (i,)),
        out_shape=jax.ShapeDtypeStruct((BATCH,), jnp.int32),
    )(x.T)

    mesh = plsc.VectorSubcoreMesh(core_axis_name="c", subcore_axis_name="s")
    run = pl.kernel(
        _gather_body,
        out_type=jax.ShapeDtypeStruct((BATCH,), jnp.float32),
        mesh=mesh,
        compiler_params=pltpu.CompilerParams(needs_layout_passes=False),
        scratch_types=[
            pltpu.VMEM((B_W,), jnp.int32),
            pltpu.VMEM((B_W,), jnp.float32),
            pltpu.SemaphoreType.DMA,
        ],
    )
    return run(indices, aux)


# speedup vs baseline: 1.7827x; 1.0116x over previous
import functools

import jax
import jax.numpy as jnp
from jax import lax
from jax.experimental import pallas as pl
from jax.experimental.pallas import tpu as pltpu
from jax.experimental.pallas import tpu_sc as plsc

L_BITS = 20
BATCH = 16384
NUM_CORES = 2
NUM_SUBCORES = 16
NUM_WORKERS = NUM_CORES * NUM_SUBCORES
B_W = BATCH // NUM_WORKERS
CHUNK = 128
N_CHUNKS = B_W // CHUNK
TC_BLOCK = 16384


def _index_body(xt_ref, idx_ref):
    xt = xt_ref[...]
    j = lax.broadcasted_iota(jnp.int32, (L_BITS, 1), 0)
    w = jnp.left_shift(jnp.int32(1), (L_BITS - 1) - j)
    contrib = jnp.where(xt > 0.0, w, jnp.zeros_like(w))
    idx_ref[...] = jnp.sum(contrib, axis=0)


def _gather_body(idx_hbm, aux_hbm, out_hbm, idx_v, out_v, sem):
    cid = lax.axis_index("c")
    sid = lax.axis_index("s")
    wid = sid * NUM_CORES + cid
    base = wid * B_W

    pltpu.sync_copy(idx_hbm.at[pl.ds(base, B_W)], idx_v)

    gathers = []
    for c in range(N_CHUNKS):
        gathers.append(
            pltpu.async_copy(
                aux_hbm.at[idx_v.at[pl.ds(c * CHUNK, CHUNK)]],
                out_v.at[pl.ds(c * CHUNK, CHUNK)],
                sem.at[c],
            )
        )
    writes = []
    for c in range(N_CHUNKS):
        gathers[c].wait()
        writes.append(
            pltpu.async_copy(
                out_v.at[pl.ds(c * CHUNK, CHUNK)],
                out_hbm.at[pl.ds(base + c * CHUNK, CHUNK)],
                sem.at[N_CHUNKS + c],
            )
        )
    for wr in writes:
        wr.wait()


@jax.jit
def kernel(x, aux):
    indices = pl.pallas_call(
        _index_body,
        grid=(BATCH // TC_BLOCK,),
        in_specs=[pl.BlockSpec((L_BITS, TC_BLOCK), lambda i: (0, i))],
        out_specs=pl.BlockSpec((TC_BLOCK,), lambda i: (i,)),
        out_shape=jax.ShapeDtypeStruct((BATCH,), jnp.int32),
    )(x.T)

    mesh = plsc.VectorSubcoreMesh(core_axis_name="c", subcore_axis_name="s")
    run = pl.kernel(
        _gather_body,
        out_type=jax.ShapeDtypeStruct((BATCH,), jnp.float32),
        mesh=mesh,
        compiler_params=pltpu.CompilerParams(needs_layout_passes=False),
        scratch_types=[
            pltpu.VMEM((B_W,), jnp.int32),
            pltpu.VMEM((B_W,), jnp.float32),
            pltpu.SemaphoreType.DMA((2 * N_CHUNKS,)),
        ],
    )
    return run(indices, aux)
